# baseline (speedup 1.0000x reference)
import jax
import jax.numpy as jnp
from jax import lax
from jax.experimental import pallas as pl
from jax.experimental.pallas import tpu as pltpu
from jax.experimental.pallas import tpu_sc as plsc

B = 4096
S = 200
D = 64
DP = 128
NC = 2
NS = 16
NW = NC * NS
BBLK = B // NW
LANES = 16
BG = BBLK // LANES
DG = D // LANES


def _embed_body(idx_hbm, table_hbm, pos_hbm, out_hbm,
                idx_v, rows0, rows1, obuf0, obuf1, pos_v,
                sem_g0, sem_g1, sem_o0, sem_o1):
    rows = (rows0, rows1)
    obuf = (obuf0, obuf1)
    sem_g = (sem_g0, sem_g1)
    sem_o = (sem_o0, sem_o1)
    wid = lax.axis_index("s") * NC + lax.axis_index("c")
    b0 = wid * BBLK
    pltpu.sync_copy(pos_hbm, pos_v)
    pltpu.sync_copy(idx_hbm.at[:, pl.ds(b0, BBLK)], idx_v)
    lane = lax.iota(jnp.int32, LANES)
    rot = [jnp.bitwise_and(lane + k, LANES - 1) for k in range(LANES)]
    ri = [j * LANES + lane for j in range(BG)]

    def fire_gather(s, ph):
        pltpu.async_copy(table_hbm.at[idx_v.at[s]], rows[ph], sem_g[ph])

    def wait_gather(s, ph):
        pltpu.make_async_copy(table_hbm.at[idx_v.at[s]], rows[ph], sem_g[ph]).wait()

    def fire_out(s, ph):
        pltpu.async_copy(obuf[ph], out_hbm.at[s, :, pl.ds(b0, BBLK)], sem_o[ph])

    def wait_out(s, ph):
        pltpu.make_async_copy(
            obuf[ph], out_hbm.at[s, :, pl.ds(b0, BBLK)], sem_o[ph]
        ).wait()

    def compute(s, ph):
        rv = rows[ph]
        ov = obuf[ph]

        @plsc.parallel_loop(0, LANES, unroll=4)
        def _(k):
            rot_k = jnp.bitwise_and(lane + k, LANES - 1)
            for dg in range(DG):
                pg = pos_v[s, pl.ds(dg * LANES, LANES)]
                ck = dg * LANES + rot_k
                pk = pg.at[rot_k].get(mode="promise_in_bounds")
                for j in range(BG):
                    vals = plsc.load_gather(rv, [ri[j], ck])
                    plsc.store_scatter(ov, [ck, ri[j]], vals + pk)

    fire_gather(0, 0)

    def s2_body(s2, carry):
        for ph in range(2):
            s = 2 * s2 + ph

            @pl.when(jnp.logical_and(s + 1 < S, s >= 1))
            def _():
                wait_out(s - 1, 1 - ph)

            @pl.when(s + 1 < S)
            def _():
                fire_gather(s + 1, 1 - ph)

            wait_gather(s, ph)
            compute(s, ph)
            fire_out(s, ph)
        return carry

    lax.fori_loop(0, S // 2, s2_body, 0)
    wait_out(S - 2, 0)
    wait_out(S - 1, 1)


def kernel(batch_seqs, item_emb, pos_weight):
    idx_t = batch_seqs.T
    table_p = jnp.pad(item_emb, ((0, 0), (0, DP - D)))
    k = pl.kernel(
        _embed_body,
        out_type=jax.ShapeDtypeStruct((S, D, B), jnp.float32),
        mesh=plsc.VectorSubcoreMesh(core_axis_name="c", subcore_axis_name="s"),
        compiler_params=pltpu.CompilerParams(
            use_tc_tiling_on_sc=True, needs_layout_passes=False
        ),
        scratch_types=[
            pltpu.VMEM((S, BBLK), jnp.int32),
            pltpu.VMEM((BBLK, DP), jnp.float32),
            pltpu.VMEM((BBLK, DP), jnp.float32),
            pltpu.VMEM((D, BBLK), jnp.float32),
            pltpu.VMEM((D, BBLK), jnp.float32),
            pltpu.VMEM((S, D), jnp.float32),
            pltpu.SemaphoreType.DMA,
            pltpu.SemaphoreType.DMA,
            pltpu.SemaphoreType.DMA,
            pltpu.SemaphoreType.DMA,
        ],
    )
    out_phys = k(idx_t, table_p, pos_weight)
    return jnp.transpose(out_phys, (2, 0, 1))


# speedup vs baseline: 1.3529x; 1.3529x over previous
import jax
import jax.numpy as jnp
from jax import lax
from jax.experimental import pallas as pl
from jax.experimental.pallas import tpu as pltpu
from jax.experimental.pallas import tpu_sc as plsc

B = 4096
S = 200
D = 64
DP = 128
NC = 2
NS = 16
NW = NC * NS
BBLK = B // NW
LANES = 16
BG = BBLK // LANES
DG = D // LANES


def _embed_body(idx_hbm, table_hbm, pos_hbm, out_hbm,
                idx_v, rows0, rows1, obuf0, obuf1, pos_v,
                sem_g0, sem_g1, sem_o0, sem_o1):
    rows = (rows0, rows1)
    obuf = (obuf0, obuf1)
    sem_g = (sem_g0, sem_g1)
    sem_o = (sem_o0, sem_o1)
    wid = lax.axis_index("s") * NC + lax.axis_index("c")
    b0 = wid * BBLK
    pltpu.sync_copy(pos_hbm, pos_v)
    pltpu.sync_copy(idx_hbm.at[:, pl.ds(b0, BBLK)], idx_v)
    lane = lax.iota(jnp.int32, LANES)
    rot = [jnp.bitwise_and(lane + k, LANES - 1) for k in range(LANES)]
    ri = [j * LANES + lane for j in range(BG)]

    def fire_gather(s, ph):
        pltpu.async_copy(table_hbm.at[idx_v.at[s]], rows[ph], sem_g[ph])

    def wait_gather(s, ph):
        pltpu.make_async_copy(table_hbm.at[idx_v.at[s]], rows[ph], sem_g[ph]).wait()

    def fire_out(s, ph):
        pltpu.async_copy(obuf[ph], out_hbm.at[s, :, pl.ds(b0, BBLK)], sem_o[ph])

    def wait_out(s, ph):
        pltpu.make_async_copy(
            obuf[ph], out_hbm.at[s, :, pl.ds(b0, BBLK)], sem_o[ph]
        ).wait()

    def compute(s, ph):
        rv = rows[ph]
        ov = obuf[ph]

        @plsc.parallel_loop(0, D, unroll=2)
        def _(i):
            dg = lax.shift_right_logical(i, 4)
            k = jnp.bitwise_and(i, LANES - 1)
            rot_k = jnp.bitwise_and(lane + k, LANES - 1)
            pg = pos_v[s, pl.ds(dg * LANES, LANES)]
            ck = dg * LANES + rot_k
            pk = pg.at[rot_k].get(mode="promise_in_bounds")
            for j in range(BG):
                vals = plsc.load_gather(rv, [ri[j], ck])
                plsc.store_scatter(ov, [ck, ri[j]], vals + pk)

    fire_gather(0, 0)

    def s2_body(s2, carry):
        for ph in range(2):
            s = 2 * s2 + ph

            @pl.when(jnp.logical_and(s + 1 < S, s >= 1))
            def _():
                wait_out(s - 1, 1 - ph)

            @pl.when(s + 1 < S)
            def _():
                fire_gather(s + 1, 1 - ph)

            wait_gather(s, ph)
            compute(s, ph)
            fire_out(s, ph)
        return carry

    lax.fori_loop(0, S // 2, s2_body, 0)
    wait_out(S - 2, 0)
    wait_out(S - 1, 1)


def kernel(batch_seqs, item_emb, pos_weight):
    idx_t = batch_seqs.T
    table_p = jnp.pad(item_emb, ((0, 0), (0, DP - D)))
    k = pl.kernel(
        _embed_body,
        out_type=jax.ShapeDtypeStruct((S, D, B), jnp.float32),
        mesh=plsc.VectorSubcoreMesh(core_axis_name="c", subcore_axis_name="s"),
        compiler_params=pltpu.CompilerParams(
            use_tc_tiling_on_sc=True, needs_layout_passes=False
        ),
        scratch_types=[
            pltpu.VMEM((S, BBLK), jnp.int32),
            pltpu.VMEM((BBLK, DP), jnp.float32),
            pltpu.VMEM((BBLK, DP), jnp.float32),
            pltpu.VMEM((D, BBLK), jnp.float32),
            pltpu.VMEM((D, BBLK), jnp.float32),
            pltpu.VMEM((S, D), jnp.float32),
            pltpu.SemaphoreType.DMA,
            pltpu.SemaphoreType.DMA,
            pltpu.SemaphoreType.DMA,
            pltpu.SemaphoreType.DMA,
        ],
    )
    out_phys = k(idx_t, table_p, pos_weight)
    return jnp.transpose(out_phys, (2, 0, 1))


# speedup vs baseline: 1.3541x; 1.0009x over previous
import jax
import jax.numpy as jnp
from jax import lax
from jax.experimental import pallas as pl
from jax.experimental.pallas import tpu as pltpu
from jax.experimental.pallas import tpu_sc as plsc

B = 4096
S = 200
D = 64
DP = 128
NC = 2
NS = 16
NW = NC * NS
BBLK = B // NW
LANES = 16
BG = BBLK // LANES
DG = D // LANES


def _embed_body(idx_hbm, table_hbm, pos_hbm, out_hbm,
                idx_v, rows0, rows1, obuf0, obuf1, pos_v,
                sem_g0, sem_g1, sem_o0, sem_o1):
    rows = (rows0, rows1)
    obuf = (obuf0, obuf1)
    sem_g = (sem_g0, sem_g1)
    sem_o = (sem_o0, sem_o1)
    wid = lax.axis_index("s") * NC + lax.axis_index("c")
    b0 = wid * BBLK
    pltpu.sync_copy(pos_hbm, pos_v)
    pltpu.sync_copy(idx_hbm.at[:, pl.ds(b0, BBLK)], idx_v)
    lane = lax.iota(jnp.int32, LANES)
    rot = [jnp.bitwise_and(lane + k, LANES - 1) for k in range(LANES)]
    ri = [j * LANES + lane for j in range(BG)]

    def fire_gather(s, ph):
        pltpu.async_copy(table_hbm.at[idx_v.at[s]], rows[ph], sem_g[ph])

    def wait_gather(s, ph):
        pltpu.make_async_copy(table_hbm.at[idx_v.at[s]], rows[ph], sem_g[ph]).wait()

    def fire_out(s, ph):
        pltpu.async_copy(obuf[ph], out_hbm.at[s, :, pl.ds(b0, BBLK)], sem_o[ph])

    def wait_out(s, ph):
        pltpu.make_async_copy(
            obuf[ph], out_hbm.at[s, :, pl.ds(b0, BBLK)], sem_o[ph]
        ).wait()

    def compute(s, ph):
        rv = rows[ph]
        ov = obuf[ph]

        @plsc.parallel_loop(0, D, unroll=4)
        def _(i):
            dg = lax.shift_right_logical(i, 4)
            k = jnp.bitwise_and(i, LANES - 1)
            rot_k = jnp.bitwise_and(lane + k, LANES - 1)
            pg = pos_v[s, pl.ds(dg * LANES, LANES)]
            ck = dg * LANES + rot_k
            pk = pg.at[rot_k].get(mode="promise_in_bounds")
            for j in range(BG):
                vals = plsc.load_gather(rv, [ri[j], ck])
                plsc.store_scatter(ov, [ck, ri[j]], vals + pk)

    fire_gather(0, 0)

    def s2_body(s2, carry):
        for ph in range(2):
            s = 2 * s2 + ph

            @pl.when(jnp.logical_and(s + 1 < S, s >= 1))
            def _():
                wait_out(s - 1, 1 - ph)

            @pl.when(s + 1 < S)
            def _():
                fire_gather(s + 1, 1 - ph)

            wait_gather(s, ph)
            compute(s, ph)
            fire_out(s, ph)
        return carry

    lax.fori_loop(0, S // 2, s2_body, 0)
    wait_out(S - 2, 0)
    wait_out(S - 1, 1)


def kernel(batch_seqs, item_emb, pos_weight):
    idx_t = batch_seqs.T
    table_p = jnp.pad(item_emb, ((0, 0), (0, DP - D)))
    k = pl.kernel(
        _embed_body,
        out_type=jax.ShapeDtypeStruct((S, D, B), jnp.float32),
        mesh=plsc.VectorSubcoreMesh(core_axis_name="c", subcore_axis_name="s"),
        compiler_params=pltpu.CompilerParams(
            use_tc_tiling_on_sc=True, needs_layout_passes=False
        ),
        scratch_types=[
            pltpu.VMEM((S, BBLK), jnp.int32),
            pltpu.VMEM((BBLK, DP), jnp.float32),
            pltpu.VMEM((BBLK, DP), jnp.float32),
            pltpu.VMEM((D, BBLK), jnp.float32),
            pltpu.VMEM((D, BBLK), jnp.float32),
            pltpu.VMEM((S, D), jnp.float32),
            pltpu.SemaphoreType.DMA,
            pltpu.SemaphoreType.DMA,
            pltpu.SemaphoreType.DMA,
            pltpu.SemaphoreType.DMA,
        ],
    )
    out_phys = k(idx_t, table_p, pos_weight)
    return jnp.transpose(out_phys, (2, 0, 1))


# speedup vs baseline: 1.4990x; 1.1070x over previous
import jax
import jax.numpy as jnp
from jax import lax
from jax.experimental import pallas as pl
from jax.experimental.pallas import tpu as pltpu
from jax.experimental.pallas import tpu_sc as plsc

B = 4096
S = 200
D = 64
NC = 2
NS = 16
NW = NC * NS
BBLK = B // NW
LANES = 16
BG = BBLK // LANES
DT = D // 8
BT = B // 128


def _embed_body(idx_hbm, table_hbm, pos_hbm, out_hbm,
                idx_v, rows0, rows1, obuf0, obuf1, pos_v,
                sem_g0, sem_g1, sem_o0, sem_o1):
    rows = (rows0, rows1)
    obuf = (obuf0, obuf1)
    sem_g = (sem_g0, sem_g1)
    sem_o = (sem_o0, sem_o1)
    wid = lax.axis_index("s") * NC + lax.axis_index("c")
    b0 = wid * BBLK
    pltpu.sync_copy(pos_hbm, pos_v)
    pltpu.sync_copy(idx_hbm.at[:, pl.ds(b0, BBLK)], idx_v)
    lane = lax.iota(jnp.int32, LANES)
    ri = [j * LANES + lane for j in range(BG)]

    def fire_gather(s, ph):
        pltpu.async_copy(table_hbm.at[idx_v.at[s]], rows[ph], sem_g[ph])

    def wait_gather(s, ph):
        pltpu.make_async_copy(table_hbm.at[idx_v.at[s]], rows[ph], sem_g[ph]).wait()

    def fire_out(s, ph):
        pltpu.async_copy(obuf[ph], out_hbm.at[s, :, wid], sem_o[ph])

    def wait_out(s, ph):
        pltpu.make_async_copy(obuf[ph], out_hbm.at[s, :, wid], sem_o[ph]).wait()

    def compute(s, ph):
        rv = rows[ph]
        ov = obuf[ph]

        @plsc.parallel_loop(0, D, unroll=2)
        def _(i):
            dg = lax.shift_right_logical(i, 4)
            k = jnp.bitwise_and(i, LANES - 1)
            rot_k = jnp.bitwise_and(lane + k, LANES - 1)
            pg = pos_v[s, pl.ds(dg * LANES, LANES)]
            ck = dg * LANES + rot_k
            ckt = lax.shift_right_logical(ck, 3)
            ckr = jnp.bitwise_and(ck, 7)
            pk = pg.at[rot_k].get(mode="promise_in_bounds")
            for j in range(BG):
                vals = plsc.load_gather(rv, [ri[j], ck])
                plsc.store_scatter(ov, [ckt, ckr, ri[j]], vals + pk)

    fire_gather(0, 0)

    def s2_body(s2, carry):
        for ph in range(2):
            s = 2 * s2 + ph

            @pl.when(jnp.logical_and(s + 1 < S, s >= 1))
            def _():
                wait_out(s - 1, 1 - ph)

            @pl.when(s + 1 < S)
            def _():
                fire_gather(s + 1, 1 - ph)

            wait_gather(s, ph)
            compute(s, ph)
            fire_out(s, ph)
        return carry

    lax.fori_loop(0, S // 2, s2_body, 0)
    wait_out(S - 2, 0)
    wait_out(S - 1, 1)


def kernel(batch_seqs, item_emb, pos_weight):
    idx_t = batch_seqs.T
    k = pl.kernel(
        _embed_body,
        out_type=jax.ShapeDtypeStruct((S, DT, BT, 8, 128), jnp.float32),
        mesh=plsc.VectorSubcoreMesh(core_axis_name="c", subcore_axis_name="s"),
        compiler_params=pltpu.CompilerParams(
            use_tc_tiling_on_sc=False, needs_layout_passes=False
        ),
        scratch_types=[
            pltpu.VMEM((S, BBLK), jnp.int32),
            pltpu.VMEM((BBLK, D), jnp.float32),
            pltpu.VMEM((BBLK, D), jnp.float32),
            pltpu.VMEM((DT, 8, BBLK), jnp.float32),
            pltpu.VMEM((DT, 8, BBLK), jnp.float32),
            pltpu.VMEM((S, D), jnp.float32),
            pltpu.SemaphoreType.DMA,
            pltpu.SemaphoreType.DMA,
            pltpu.SemaphoreType.DMA,
            pltpu.SemaphoreType.DMA,
        ],
    )
    out5 = k(idx_t, item_emb, pos_weight)
    y = jnp.transpose(out5, (2, 4, 0, 1, 3))
    return y.reshape(B, S, D)


# speedup vs baseline: 1.5106x; 1.0078x over previous
import jax
import jax.numpy as jnp
from jax import lax
from jax.experimental import pallas as pl
from jax.experimental.pallas import tpu as pltpu
from jax.experimental.pallas import tpu_sc as plsc

B = 4096
S = 200
D = 64
NC = 2
NS = 16
NW = NC * NS
BBLK = B // NW
LANES = 16
BG = BBLK // LANES
DT = D // 8
BT = B // 128


def _embed_body(idx_hbm, table_hbm, pos_hbm, out_hbm,
                idx_v, rows0, rows1, obuf0, obuf1, pos_v,
                sem_g0, sem_g1, sem_o0, sem_o1):
    rows = (rows0, rows1)
    obuf = (obuf0, obuf1)
    sem_g = (sem_g0, sem_g1)
    sem_o = (sem_o0, sem_o1)
    wid = lax.axis_index("s") * NC + lax.axis_index("c")
    pltpu.sync_copy(pos_hbm, pos_v)
    pltpu.sync_copy(idx_hbm.at[:, wid], idx_v)
    lane = lax.iota(jnp.int32, LANES)
    ri = [j * LANES + lane for j in range(BG)]

    def idx_row(s):
        return idx_v.at[lax.shift_right_logical(s, 3), jnp.bitwise_and(s, 7)]

    def fire_gather(s, ph):
        pltpu.async_copy(table_hbm.at[idx_row(s)], rows[ph], sem_g[ph])

    def wait_gather(s, ph):
        pltpu.make_async_copy(table_hbm.at[idx_row(s)], rows[ph], sem_g[ph]).wait()

    def fire_out(s, ph):
        pltpu.async_copy(obuf[ph], out_hbm.at[s, :, wid], sem_o[ph])

    def wait_out(s, ph):
        pltpu.make_async_copy(obuf[ph], out_hbm.at[s, :, wid], sem_o[ph]).wait()

    def compute(s, ph):
        rv = rows[ph]
        ov = obuf[ph]

        @plsc.parallel_loop(0, D, unroll=2)
        def _(i):
            dg = lax.shift_right_logical(i, 4)
            k = jnp.bitwise_and(i, LANES - 1)
            rot_k = jnp.bitwise_and(lane + k, LANES - 1)
            pg = pos_v[s, pl.ds(dg * LANES, LANES)]
            ck = dg * LANES + rot_k
            ckt = lax.shift_right_logical(ck, 3)
            ckr = jnp.bitwise_and(ck, 7)
            pk = pg.at[rot_k].get(mode="promise_in_bounds")
            for j in range(BG):
                vals = plsc.load_gather(rv, [ri[j], ck])
                plsc.store_scatter(ov, [ckt, ckr, ri[j]], vals + pk)

    fire_gather(0, 0)

    def s2_body(s2, carry):
        for ph in range(2):
            s = 2 * s2 + ph

            @pl.when(jnp.logical_and(s + 1 < S, s >= 1))
            def _():
                wait_out(s - 1, 1 - ph)

            @pl.when(s + 1 < S)
            def _():
                fire_gather(s + 1, 1 - ph)

            wait_gather(s, ph)
            compute(s, ph)
            fire_out(s, ph)
        return carry

    lax.fori_loop(0, S // 2, s2_body, 0)
    wait_out(S - 2, 0)
    wait_out(S - 1, 1)


def kernel(batch_seqs, item_emb, pos_weight):
    idx_t = batch_seqs.T.reshape(S // 8, 8, BT, 128).transpose(0, 2, 1, 3)
    k = pl.kernel(
        _embed_body,
        out_type=jax.ShapeDtypeStruct((S, DT, BT, 8, 128), jnp.float32),
        mesh=plsc.VectorSubcoreMesh(core_axis_name="c", subcore_axis_name="s"),
        compiler_params=pltpu.CompilerParams(
            use_tc_tiling_on_sc=False, needs_layout_passes=False
        ),
        scratch_types=[
            pltpu.VMEM((S // 8, 8, BBLK), jnp.int32),
            pltpu.VMEM((BBLK, D), jnp.float32),
            pltpu.VMEM((BBLK, D), jnp.float32),
            pltpu.VMEM((DT, 8, BBLK), jnp.float32),
            pltpu.VMEM((DT, 8, BBLK), jnp.float32),
            pltpu.VMEM((S, D), jnp.float32),
            pltpu.SemaphoreType.DMA,
            pltpu.SemaphoreType.DMA,
            pltpu.SemaphoreType.DMA,
            pltpu.SemaphoreType.DMA,
        ],
    )
    out5 = k(idx_t, item_emb, pos_weight)
    y = jnp.transpose(out5, (2, 4, 0, 1, 3))
    return y.reshape(B, S, D)
